# baseline (speedup 1.0000x reference)
import functools

import jax
import jax.numpy as jnp
from jax import lax
from jax.experimental import pallas as pl
from jax.experimental.pallas import tpu as pltpu
from jax.experimental.pallas import tpu_sc as plsc

B = 16384
VOCAB = 1000
EMB = 16
ND = 13
H = 16
NIN = ND + 2 * EMB

NC = 2
NS = 16
NW = NC * NS
BPW = B // NW
L = 16
RPC = 2 * L
NCHK = BPW // RPC

W2_OFF = NIN * H
B1_OFF = W2_OFF + H
NSPLAT = B1_OFF + H

_mesh = plsc.VectorSubcoreMesh(core_axis_name="c", subcore_axis_name="s")


@functools.partial(
    pl.kernel,
    mesh=_mesh,
    compiler_params=pltpu.CompilerParams(use_tc_tiling_on_sc=False,
                                         needs_layout_passes=False),
    out_type=jax.ShapeDtypeStruct((B,), jnp.float32),
    scratch_types=[
        pltpu.VMEM((BPW * ND,), jnp.float32),
        pltpu.VMEM((BPW,), jnp.int32),
        pltpu.VMEM((BPW,), jnp.int32),
        pltpu.VMEM((VOCAB * EMB,), jnp.float32),
        pltpu.VMEM((VOCAB * EMB,), jnp.float32),
        pltpu.VMEM((NIN * H + 8,), jnp.float32),
        pltpu.VMEM((H + 8,), jnp.float32),
        pltpu.VMEM((H + 8,), jnp.float32),
        pltpu.VMEM((16,), jnp.float32),
        pltpu.VMEM((NSPLAT * L,), jnp.float32),
        pltpu.VMEM((BPW,), jnp.float32),
        pltpu.SemaphoreType.DMA,
    ],
)
def _sc_fused(i1_hbm, c1_hbm, c2_hbm, emb1_hbm, emb2_hbm, w1_hbm, b1_hbm,
              w2_hbm, b2_hbm, out_hbm,
              i1_v, c1_v, c2_v, e1_v, e2_v, w1_vm, b1_vm, w2_vm, b2_vm,
              wsp, out_v, sem):
    wid = lax.axis_index("s") * NC + lax.axis_index("c")
    base = wid * BPW
    cps = [
        pltpu.async_copy(i1_hbm.at[pl.ds(base * ND, BPW * ND)], i1_v, sem),
        pltpu.async_copy(c1_hbm.at[pl.ds(base, BPW)], c1_v, sem),
        pltpu.async_copy(c2_hbm.at[pl.ds(base, BPW)], c2_v, sem),
        pltpu.async_copy(emb1_hbm, e1_v, sem),
        pltpu.async_copy(emb2_hbm, e2_v, sem),
        pltpu.async_copy(w1_hbm, w1_vm.at[pl.ds(8, NIN * H)], sem),
        pltpu.async_copy(b1_hbm, b1_vm.at[pl.ds(8, H)], sem),
        pltpu.async_copy(w2_hbm, w2_vm.at[pl.ds(8, H)], sem),
        pltpu.async_copy(b2_hbm, b2_vm.at[pl.ds(8, 1)], sem),
    ]
    for cp in cps:
        cp.wait()

    lanes = lax.iota(jnp.int32, L)
    zeros = jnp.zeros((L,), jnp.int32)
    for i in range(NIN * H):
        wsp[pl.ds(i * L, L)] = plsc.load_gather(w1_vm, [zeros + (8 + i)])
    for j in range(H):
        wsp[pl.ds((W2_OFF + j) * L, L)] = plsc.load_gather(
            w2_vm, [zeros + (8 + j)])
        wsp[pl.ds((B1_OFF + j) * L, L)] = plsc.load_gather(
            b1_vm, [zeros + (8 + j)])
    b2vec = plsc.load_gather(b2_vm, [zeros + 8])

    def chunk(c, _):
        row0 = c * RPC
        rows_a = lanes + row0
        rows_b = rows_a + L
        rb13a = rows_a * ND
        rb13b = rows_b * ND
        c1a = c1_v[pl.ds(row0, L)] * EMB
        c1b = c1_v[pl.ds(row0 + L, L)] * EMB
        c2a = c2_v[pl.ds(row0, L)] * EMB
        c2b = c2_v[pl.ds(row0 + L, L)] * EMB
        binit = [wsp[pl.ds((B1_OFF + j) * L, L)] for j in range(H)]
        acca = list(binit)
        accb = list(binit)

        def fma_block(cola, colb, woff, acca, accb):
            for j in range(H):
                w = wsp[pl.ds((woff + j) * L, L)]
                acca[j] = acca[j] + cola * w
                accb[j] = accb[j] + colb * w

        for k in range(ND):
            fma_block(plsc.load_gather(i1_v, [rb13a + k]),
                      plsc.load_gather(i1_v, [rb13b + k]), k * H, acca, accb)
        for jp in range(EMB):
            fma_block(plsc.load_gather(e1_v, [c1a + jp]),
                      plsc.load_gather(e1_v, [c1b + jp]),
                      (ND + jp) * H, acca, accb)
        for jp in range(EMB):
            fma_block(plsc.load_gather(e2_v, [c2a + jp]),
                      plsc.load_gather(e2_v, [c2b + jp]),
                      (ND + EMB + jp) * H, acca, accb)
        outa = b2vec
        outb = b2vec
        for j in range(H):
            w2j = wsp[pl.ds((W2_OFF + j) * L, L)]
            outa = outa + jnp.maximum(acca[j], 0.0) * w2j
            outb = outb + jnp.maximum(accb[j], 0.0) * w2j
        out_v[pl.ds(row0, L)] = outa
        out_v[pl.ds(row0 + L, L)] = outb
        return ()

    lax.fori_loop(0, NCHK, chunk, (), unroll=2)
    pltpu.sync_copy(out_v, out_hbm.at[pl.ds(base, BPW)])


def kernel(I1, C1, C2, emb1, emb2, W1, b1, W2, b2):
    out = _sc_fused(
        I1.reshape(B * ND),
        C1.astype(jnp.int32).reshape(B),
        C2.astype(jnp.int32).reshape(B),
        emb1.reshape(VOCAB * EMB),
        emb2.reshape(VOCAB * EMB),
        W1.reshape(NIN * H), b1, W2.reshape(H), b2)
    return out.reshape(B, 1)


# speedup vs baseline: 1.1430x; 1.1430x over previous
import functools

import jax
import jax.numpy as jnp
from jax import lax
from jax.experimental import pallas as pl
from jax.experimental.pallas import tpu as pltpu
from jax.experimental.pallas import tpu_sc as plsc

B = 16384
VOCAB = 1000
EMB = 16
ND = 13
H = 16
NIN = ND + 2 * EMB

NC = 2
NS = 16
NW = NC * NS
BPW = B // NW
L = 16
RPC = 2 * L
NCHK = BPW // RPC

W2_OFF = NIN * H
B1_OFF = W2_OFF + H
NSPLAT = B1_OFF + H
VOCAB_PAD = 1024
VPT = VOCAB_PAD // NS

_mesh = plsc.VectorSubcoreMesh(core_axis_name="c", subcore_axis_name="s")


@functools.partial(
    pl.kernel,
    mesh=_mesh,
    compiler_params=pltpu.CompilerParams(use_tc_tiling_on_sc=False,
                                         needs_layout_passes=False),
    out_type=jax.ShapeDtypeStruct((B,), jnp.float32),
    scratch_types=[
        pltpu.VMEM((BPW * ND,), jnp.float32),
        pltpu.VMEM((BPW,), jnp.int32),
        pltpu.VMEM((BPW,), jnp.int32),
        pltpu.VMEM((VOCAB * EMB,), jnp.float32),
        pltpu.VMEM((VOCAB * EMB,), jnp.float32),
        pltpu.VMEM((NIN * H + 8,), jnp.float32),
        pltpu.VMEM((H + 8,), jnp.float32),
        pltpu.VMEM((H + 8,), jnp.float32),
        pltpu.VMEM((16,), jnp.float32),
        pltpu.VMEM((NSPLAT * L,), jnp.float32),
        pltpu.VMEM((VOCAB_PAD * EMB,), jnp.float32),
        pltpu.VMEM((VOCAB_PAD * EMB,), jnp.float32),
        pltpu.VMEM_SHARED((VOCAB_PAD * EMB,), jnp.float32),
        pltpu.VMEM_SHARED((VOCAB_PAD * EMB,), jnp.float32),
        pltpu.VMEM((BPW,), jnp.float32),
        pltpu.SemaphoreType.DMA,
    ],
)
def _sc_fused(i1_hbm, c1_hbm, c2_hbm, emb1_hbm, emb2_hbm, w1_hbm, b1_hbm,
              w2_hbm, b2_hbm, out_hbm,
              i1_v, c1_v, c2_v, e1_v, e2_v, w1_vm, b1_vm, w2_vm, b2_vm,
              wsp, t1_v, t2_v, sp1, sp2, out_v, sem):
    wid = lax.axis_index("s") * NC + lax.axis_index("c")
    base = wid * BPW
    cps = [
        pltpu.async_copy(i1_hbm.at[pl.ds(base * ND, BPW * ND)], i1_v, sem),
        pltpu.async_copy(c1_hbm.at[pl.ds(base, BPW)], c1_v, sem),
        pltpu.async_copy(c2_hbm.at[pl.ds(base, BPW)], c2_v, sem),
        pltpu.async_copy(emb1_hbm, e1_v, sem),
        pltpu.async_copy(emb2_hbm, e2_v, sem),
        pltpu.async_copy(w1_hbm, w1_vm.at[pl.ds(8, NIN * H)], sem),
        pltpu.async_copy(b1_hbm, b1_vm.at[pl.ds(8, H)], sem),
        pltpu.async_copy(w2_hbm, w2_vm.at[pl.ds(8, H)], sem),
        pltpu.async_copy(b2_hbm, b2_vm.at[pl.ds(8, 1)], sem),
    ]
    for cp in cps:
        cp.wait()

    lanes = lax.iota(jnp.int32, L)
    zeros = jnp.zeros((L,), jnp.int32)
    for i in range(NIN * H):
        wsp[pl.ds(i * L, L)] = plsc.load_gather(w1_vm, [zeros + (8 + i)])
    for j in range(H):
        wsp[pl.ds((W2_OFF + j) * L, L)] = plsc.load_gather(
            w2_vm, [zeros + (8 + j)])
        wsp[pl.ds((B1_OFF + j) * L, L)] = plsc.load_gather(
            b1_vm, [zeros + (8 + j)])
    b2vec = plsc.load_gather(b2_vm, [zeros + 8])

    sid = lax.axis_index("s")
    vs = jnp.minimum(sid * VPT, VOCAB - VPT)
    for e_v, t_v, woff0 in ((e1_v, t1_v, ND), (e2_v, t2_v, ND + EMB)):
        for fc in range(VPT // L):
            vrows = lanes + (vs + fc * L)
            vbase = vrows * EMB
            ecols = [plsc.load_gather(e_v, [vbase + jp]) for jp in range(EMB)]
            for j in range(H):
                t = ecols[0] * wsp[pl.ds(((woff0) * H + j) * L, L)]
                for jp in range(1, EMB):
                    t = t + ecols[jp] * wsp[
                        pl.ds(((woff0 + jp) * H + j) * L, L)]
                plsc.store_scatter(t_v, [vbase + j], t)
    pltpu.sync_copy(t1_v.at[pl.ds(vs * EMB, VPT * EMB)],
                    sp1.at[pl.ds(vs * EMB, VPT * EMB)])
    pltpu.sync_copy(t2_v.at[pl.ds(vs * EMB, VPT * EMB)],
                    sp2.at[pl.ds(vs * EMB, VPT * EMB)])
    plsc.subcore_barrier()
    pltpu.sync_copy(sp1, t1_v)
    pltpu.sync_copy(sp2, t2_v)

    def chunk(c, _):
        row0 = c * RPC
        rows_a = lanes + row0
        rows_b = rows_a + L
        rb13a = rows_a * ND
        rb13b = rows_b * ND
        c1a = c1_v[pl.ds(row0, L)] * EMB
        c1b = c1_v[pl.ds(row0 + L, L)] * EMB
        c2a = c2_v[pl.ds(row0, L)] * EMB
        c2b = c2_v[pl.ds(row0 + L, L)] * EMB
        binit = [wsp[pl.ds((B1_OFF + j) * L, L)] for j in range(H)]
        acca = list(binit)
        accb = list(binit)

        def fma_block(cola, colb, woff, acca, accb):
            for j in range(H):
                w = wsp[pl.ds((woff + j) * L, L)]
                acca[j] = acca[j] + cola * w
                accb[j] = accb[j] + colb * w

        for k in range(ND):
            fma_block(plsc.load_gather(i1_v, [rb13a + k]),
                      plsc.load_gather(i1_v, [rb13b + k]), k * H, acca, accb)
        for j in range(H):
            acca[j] = (acca[j] + plsc.load_gather(t1_v, [c1a + j])
                       + plsc.load_gather(t2_v, [c2a + j]))
            accb[j] = (accb[j] + plsc.load_gather(t1_v, [c1b + j])
                       + plsc.load_gather(t2_v, [c2b + j]))
        outa = b2vec
        outb = b2vec
        for j in range(H):
            w2j = wsp[pl.ds((W2_OFF + j) * L, L)]
            outa = outa + jnp.maximum(acca[j], 0.0) * w2j
            outb = outb + jnp.maximum(accb[j], 0.0) * w2j
        out_v[pl.ds(row0, L)] = outa
        out_v[pl.ds(row0 + L, L)] = outb
        return ()

    lax.fori_loop(0, NCHK, chunk, (), unroll=False)
    pltpu.sync_copy(out_v, out_hbm.at[pl.ds(base, BPW)])


def kernel(I1, C1, C2, emb1, emb2, W1, b1, W2, b2):
    out = _sc_fused(
        I1.reshape(B * ND),
        C1.astype(jnp.int32).reshape(B),
        C2.astype(jnp.int32).reshape(B),
        emb1.reshape(VOCAB * EMB),
        emb2.reshape(VOCAB * EMB),
        W1.reshape(NIN * H), b1, W2.reshape(H), b2)
    return out.reshape(B, 1)


# speedup vs baseline: 1.2651x; 1.1069x over previous
import functools

import jax
import jax.numpy as jnp
from jax import lax
from jax.experimental import pallas as pl
from jax.experimental.pallas import tpu as pltpu
from jax.experimental.pallas import tpu_sc as plsc

B = 16384
VOCAB = 1000
EMB = 16
ND = 13

NC = 2
NS = 16
NW = NC * NS
BPW = B // NW
CH = 128
NCH = BPW // CH

_mesh = plsc.VectorSubcoreMesh(core_axis_name="c", subcore_axis_name="s")


@functools.partial(
    pl.kernel,
    mesh=_mesh,
    compiler_params=pltpu.CompilerParams(use_tc_tiling_on_sc=False),
    out_type=(
        jax.ShapeDtypeStruct((B, EMB), jnp.float32),
        jax.ShapeDtypeStruct((B, EMB), jnp.float32),
    ),
    scratch_types=[
        pltpu.VMEM((NCH, CH), jnp.int32),
        pltpu.VMEM((NCH, CH), jnp.int32),
        pltpu.VMEM((BPW, EMB), jnp.float32),
        pltpu.VMEM((BPW, EMB), jnp.float32),
        pltpu.SemaphoreType.DMA,
        pltpu.SemaphoreType.DMA,
    ],
)
def _sc_gather(emb1_hbm, emb2_hbm, c1_hbm, c2_hbm, o1_hbm, o2_hbm,
               idx1, idx2, r1, r2, sem1, sem2):
    wid = lax.axis_index("s") * NC + lax.axis_index("c")
    base = wid * BPW
    pltpu.sync_copy(c1_hbm.at[wid], idx1)
    pltpu.sync_copy(c2_hbm.at[wid], idx2)
    cps = []
    for k in range(NCH):
        cps.append(pltpu.async_copy(
            emb1_hbm.at[idx1.at[k]], r1.at[pl.ds(k * CH, CH)], sem1))
        cps.append(pltpu.async_copy(
            emb2_hbm.at[idx2.at[k]], r2.at[pl.ds(k * CH, CH)], sem2))
    for cp in cps:
        cp.wait()
    pltpu.sync_copy(r1, o1_hbm.at[pl.ds(base, BPW)])
    pltpu.sync_copy(r2, o2_hbm.at[pl.ds(base, BPW)])


def _mlp_body(i1_ref, e1_ref, e2_ref, w1_ref, b1_ref, w2_ref, b2_ref, o_ref):
    w1 = w1_ref[...]
    h = jnp.dot(i1_ref[...], w1[:ND, :], preferred_element_type=jnp.float32)
    h = h + jnp.dot(e1_ref[...], w1[ND:ND + EMB, :],
                    preferred_element_type=jnp.float32)
    h = h + jnp.dot(e2_ref[...], w1[ND + EMB:, :],
                    preferred_element_type=jnp.float32)
    h = jnp.maximum(h + b1_ref[...], 0.0)
    o_ref[...] = jnp.sum(h * w2_ref[...], axis=1, keepdims=True) + b2_ref[...]


BLK = 16384


def _mlp_call(I1, e1g, e2g, W1, b1r, W2r, b2r):
    grid = (B // BLK,)
    return pl.pallas_call(
        _mlp_body,
        grid=grid,
        in_specs=[
            pl.BlockSpec((BLK, ND), lambda i: (i, 0)),
            pl.BlockSpec((BLK, EMB), lambda i: (i, 0)),
            pl.BlockSpec((BLK, EMB), lambda i: (i, 0)),
            pl.BlockSpec((ND + 2 * EMB, 16), lambda i: (0, 0)),
            pl.BlockSpec((1, 16), lambda i: (0, 0)),
            pl.BlockSpec((1, 16), lambda i: (0, 0)),
            pl.BlockSpec((1, 1), lambda i: (0, 0)),
        ],
        out_specs=pl.BlockSpec((BLK, 1), lambda i: (i, 0)),
        out_shape=jax.ShapeDtypeStruct((B, 1), jnp.float32),
    )(I1, e1g, e2g, W1, b1r, W2r, b2r)


def kernel(I1, C1, C2, emb1, emb2, W1, b1, W2, b2):
    c1 = C1.astype(jnp.int32).reshape(NW, NCH, CH)
    c2 = C2.astype(jnp.int32).reshape(NW, NCH, CH)
    e1g, e2g = _sc_gather(emb1, emb2, c1, c2)
    return _mlp_call(I1, e1g, e2g, W1,
                     b1.reshape(1, EMB), W2.reshape(1, EMB), b2.reshape(1, 1))


# speedup vs baseline: 1.2779x; 1.0101x over previous
import functools

import jax
import jax.numpy as jnp
from jax import lax
from jax.experimental import pallas as pl
from jax.experimental.pallas import tpu as pltpu
from jax.experimental.pallas import tpu_sc as plsc

B = 16384
VOCAB = 1000
EMB = 16
ND = 13

NC = 2
NS = 16
NW = NC * NS
BPW = B // NW
CH = 128
NCH = BPW // CH

_mesh = plsc.VectorSubcoreMesh(core_axis_name="c", subcore_axis_name="s")


@functools.partial(
    pl.kernel,
    mesh=_mesh,
    compiler_params=pltpu.CompilerParams(use_tc_tiling_on_sc=False),
    out_type=(
        jax.ShapeDtypeStruct((B, EMB), jnp.float32),
        jax.ShapeDtypeStruct((B, EMB), jnp.float32),
    ),
    scratch_types=[
        pltpu.VMEM((NCH, CH), jnp.int32),
        pltpu.VMEM((NCH, CH), jnp.int32),
        pltpu.VMEM((BPW, EMB), jnp.float32),
        pltpu.VMEM((BPW, EMB), jnp.float32),
        pltpu.SemaphoreType.DMA,
        pltpu.SemaphoreType.DMA,
    ],
)
def _sc_gather(emb1_hbm, emb2_hbm, c1_hbm, c2_hbm, o1_hbm, o2_hbm,
               idx1, idx2, r1, r2, sem1, sem2):
    wid = lax.axis_index("s") * NC + lax.axis_index("c")
    base = wid * BPW
    i1cp = pltpu.async_copy(c1_hbm.at[wid], idx1, sem1)
    i2cp = pltpu.async_copy(c2_hbm.at[wid], idx2, sem2)
    i1cp.wait()
    i2cp.wait()
    cps = []
    for k in range(NCH):
        cps.append(pltpu.async_copy(
            emb1_hbm.at[idx1.at[k]], r1.at[pl.ds(k * CH, CH)], sem1))
        cps.append(pltpu.async_copy(
            emb2_hbm.at[idx2.at[k]], r2.at[pl.ds(k * CH, CH)], sem2))
    for cp in cps:
        cp.wait()
    o1cp = pltpu.async_copy(r1, o1_hbm.at[pl.ds(base, BPW)], sem1)
    o2cp = pltpu.async_copy(r2, o2_hbm.at[pl.ds(base, BPW)], sem2)
    o1cp.wait()
    o2cp.wait()


def _mlp_body(i1_ref, e1_ref, e2_ref, w1_ref, b1_ref, w2_ref, b2_ref, o_ref):
    w1 = w1_ref[...]
    h = jnp.dot(i1_ref[...], w1[:ND, :], preferred_element_type=jnp.float32)
    h = h + jnp.dot(e1_ref[...], w1[ND:ND + EMB, :],
                    preferred_element_type=jnp.float32)
    h = h + jnp.dot(e2_ref[...], w1[ND + EMB:, :],
                    preferred_element_type=jnp.float32)
    h = jnp.maximum(h + b1_ref[...], 0.0)
    o_ref[...] = jnp.sum(h * w2_ref[...], axis=1, keepdims=True) + b2_ref[...]


BLK = 2048


def _mlp_call(I1, e1g, e2g, W1, b1r, W2r, b2r):
    grid = (B // BLK,)
    return pl.pallas_call(
        _mlp_body,
        grid=grid,
        in_specs=[
            pl.BlockSpec((BLK, ND), lambda i: (i, 0)),
            pl.BlockSpec((BLK, EMB), lambda i: (i, 0)),
            pl.BlockSpec((BLK, EMB), lambda i: (i, 0)),
            pl.BlockSpec((ND + 2 * EMB, 16), lambda i: (0, 0)),
            pl.BlockSpec((1, 16), lambda i: (0, 0)),
            pl.BlockSpec((1, 16), lambda i: (0, 0)),
            pl.BlockSpec((1, 1), lambda i: (0, 0)),
        ],
        out_specs=pl.BlockSpec((BLK, 1), lambda i: (i, 0)),
        out_shape=jax.ShapeDtypeStruct((B, 1), jnp.float32),
    )(I1, e1g, e2g, W1, b1r, W2r, b2r)


def kernel(I1, C1, C2, emb1, emb2, W1, b1, W2, b2):
    c1 = C1.astype(jnp.int32).reshape(NW, NCH, CH)
    c2 = C2.astype(jnp.int32).reshape(NW, NCH, CH)
    e1g, e2g = _sc_gather(emb1, emb2, c1, c2)
    return _mlp_call(I1, e1g, e2g, W1,
                     b1.reshape(1, EMB), W2.reshape(1, EMB), b2.reshape(1, 1))
